# baseline (speedup 1.0000x reference)
import jax
import jax.numpy as jnp
from jax import lax
from jax.experimental import pallas as pl
from jax.experimental.pallas import tpu as pltpu
from jax.experimental.pallas import tpu_sc as plsc

N = 10000
NPAD = 10240
F = 128
E = 320000
NW = 32
GROUPS = 80
EPW = GROUPS * 128
EPAD = EPW * NW
RPT = NPAD // 16
NB = 1024
BPW = NB // NW

_mesh = plsc.VectorSubcoreMesh(core_axis_name="c", subcore_axis_name="s")
_f32 = jnp.float32
_bf16 = jnp.bfloat16
_s16 = jnp.int16


def _deg_body(dst_hbm, ones_hbm, zeros_hbm, out_hbm, idx_v, ones_v, acc_sh):
    c = lax.axis_index("c")
    s = lax.axis_index("s")
    wid = s * 2 + c
    pltpu.sync_copy(zeros_hbm.at[pl.ds(s * RPT, RPT)],
                    acc_sh.at[pl.ds(s * RPT, RPT)])
    pltpu.sync_copy(ones_hbm, ones_v)
    pltpu.sync_copy(dst_hbm.at[wid], idx_v)
    plsc.subcore_barrier()

    def body(g, carry):
        pltpu.sync_copy(ones_v, acc_sh.at[idx_v.at[g]], add=True)
        return carry

    lax.fori_loop(0, GROUPS, body, 0)
    plsc.subcore_barrier()
    pltpu.sync_copy(acc_sh.at[pl.ds(s * RPT, RPT)],
                    out_hbm.at[c].at[pl.ds(s * RPT, RPT)])


_deg_kernel = pl.kernel(
    _deg_body,
    out_type=jax.ShapeDtypeStruct((2, NPAD, F), _f32),
    mesh=_mesh,
    scratch_types=[
        pltpu.VMEM((GROUPS, 128), jnp.int32),
        pltpu.VMEM((128, F), _f32),
        pltpu.VMEM_SHARED((NPAD, F), _f32),
    ],
)


RING = 2


HGRP = GROUPS // 2


def _scatter_body(h_hbm, src_hbm, dst_hbm, zeros_hbm, out_hbm,
                  src_v, dst_v, r0, r1, acc_sh, gsem):
    rows = (r0, r1)
    c = lax.axis_index("c")
    s = lax.axis_index("s")
    wid = s * 2 + c
    pltpu.sync_copy(zeros_hbm.at[pl.ds(s * RPT, RPT)],
                    acc_sh.at[pl.ds(s * RPT, RPT)])
    plsc.subcore_barrier()

    def body(i, carry):
        g0 = i * RING
        descs = [pltpu.async_copy(h_hbm.at[src_v.at[g0 + k]], rows[k],
                                  gsem.at[k]) for k in range(RING)]
        for k in range(RING):
            descs[k].wait()
            pltpu.sync_copy(rows[k], acc_sh.at[dst_v.at[g0 + k]], add=True)
        return carry

    for h in range(2):
        pltpu.sync_copy(src_hbm.at[wid].at[pl.ds(h * HGRP, HGRP)], src_v)
        pltpu.sync_copy(dst_hbm.at[wid].at[pl.ds(h * HGRP, HGRP)], dst_v)
        lax.fori_loop(0, HGRP // RING, body, 0)
    plsc.subcore_barrier()
    pltpu.sync_copy(acc_sh.at[pl.ds(s * RPT, RPT)],
                    out_hbm.at[c].at[pl.ds(s * RPT, RPT)])


_scatter_kernel = pl.kernel(
    _scatter_body,
    out_type=jax.ShapeDtypeStruct((2, NPAD, F), _f32),
    mesh=_mesh,
    scratch_types=[
        pltpu.VMEM((HGRP, 128), jnp.int32),
        pltpu.VMEM((HGRP, 128), jnp.int32),
        pltpu.VMEM((128, F), _f32),
        pltpu.VMEM((128, F), _f32),
        pltpu.VMEM_SHARED((NPAD, F), _f32),
        pltpu.SemaphoreType.DMA((RING,)),
    ],
)


def _gather_body(p2_hbm, h2_hbm, dis_hbm, bn_hbm,
                 gp0_hbm, gp1_hbm, gh_hbm, gd_hbm,
                 idx_v, bp0, bp1, bh, bd, sem):
    c = lax.axis_index("c")
    s = lax.axis_index("s")
    wid = s * 2 + c
    pltpu.sync_copy(bn_hbm.at[wid], idx_v)
    pltpu.async_copy(p2_hbm.at[0].at[idx_v.at[0]], bp0, sem).wait()
    pltpu.async_copy(p2_hbm.at[1].at[idx_v.at[0]], bp1, sem).wait()
    pltpu.async_copy(h2_hbm.at[idx_v.at[0]], bh, sem).wait()
    pltpu.async_copy(dis_hbm.at[idx_v.at[0]], bd, sem).wait()
    base = wid * BPW
    pltpu.sync_copy(bp0, gp0_hbm.at[pl.ds(base, BPW)])
    pltpu.sync_copy(bp1, gp1_hbm.at[pl.ds(base, BPW)])
    pltpu.sync_copy(bh, gh_hbm.at[pl.ds(base, BPW)])
    pltpu.sync_copy(bd, gd_hbm.at[pl.ds(base, BPW)])


_gather_kernel = pl.kernel(
    _gather_body,
    out_type=(
        jax.ShapeDtypeStruct((NB, F), _f32),
        jax.ShapeDtypeStruct((NB, F), _f32),
        jax.ShapeDtypeStruct((NB, F), _f32),
        jax.ShapeDtypeStruct((NB, F), _f32),
    ),
    mesh=_mesh,
    scratch_types=[
        pltpu.VMEM((1, BPW), jnp.int32),
        pltpu.VMEM((BPW, F), _f32),
        pltpu.VMEM((BPW, F), _f32),
        pltpu.VMEM((BPW, F), _f32),
        pltpu.VMEM((BPW, F), _f32),
        pltpu.SemaphoreType.DMA,
    ],
)


def _mm_body(x_ref, w_ref, o_ref):
    o_ref[...] = jnp.dot(x_ref[...], w_ref[...],
                         preferred_element_type=jnp.float32)


_mm = pl.pallas_call(
    _mm_body,
    out_shape=jax.ShapeDtypeStruct((N, F), _f32),
)


def _prep_body(degp_ref, h_ref, dis_ref, hp_ref):
    deg = degp_ref[0, :N, 0:1] + degp_ref[1, :N, 0:1] + 1.0
    dis = lax.rsqrt(deg)
    dis_ref[...] = jnp.broadcast_to(dis, (N, F))
    hp_ref[...] = (h_ref[...] * dis).astype(_f32)


_prep = pl.pallas_call(
    _prep_body,
    out_shape=(
        jax.ShapeDtypeStruct((N, F), _f32),
        jax.ShapeDtypeStruct((N, F), _f32),
    ),
)


def _layer_body(p_ref, hp_ref, dis_ref, s1_ref, c1_ref, w_ref, o_ref):
    agg = p_ref[0, :N] + p_ref[1, :N] + hp_ref[...]
    d = dis_ref[:, 0:1]
    z = agg * d * s1_ref[...] + c1_ref[...]
    x1 = jnp.maximum(z, 0.0)
    h2 = jnp.dot(x1, w_ref[...], preferred_element_type=jnp.float32)
    o_ref[...] = (h2 * d).astype(_f32)


_layer = pl.pallas_call(
    _layer_body,
    out_shape=jax.ShapeDtypeStruct((N, F), _f32),
)


def _final_body(gp0_ref, gp1_ref, gh_ref, gd_ref, s2_ref, c2_ref, o_ref):
    agg = gp0_ref[...] + gp1_ref[...] + gh_ref[...]
    z = agg * gd_ref[:, 0:1] * s2_ref[...] + c2_ref[...]
    x2 = jnp.maximum(z, 0.0)
    m = jnp.max(x2, axis=1, keepdims=True)
    e = jnp.exp(x2 - m)
    lse = jnp.log(jnp.sum(e, axis=1, keepdims=True))
    o_ref[...] = (x2 - m) - lse


_final = pl.pallas_call(
    _final_body,
    out_shape=jax.ShapeDtypeStruct((NB, F), _f32),
)


def kernel(features, edge_index, batch_nodes, device,
           W1, b1, gamma1, beta1, rm1, rv1,
           W2, b2, gamma2, beta2, rm2, rv2):
    del device
    pad = EPAD - E
    src_p = jnp.concatenate(
        [edge_index[0], jnp.zeros((pad,), jnp.int32)]).reshape(NW, GROUPS, 128)
    dst_p = jnp.concatenate(
        [edge_index[1], jnp.full((pad,), N, jnp.int32)]).reshape(NW, GROUPS, 128)
    bn = batch_nodes.reshape(NW, 1, BPW)
    ones128 = jnp.ones((128, F), _f32)
    zeros_s = jnp.zeros((NPAD, F), _f32)
    zeros_b = jnp.zeros((NPAD, F), _f32)
    eps = 1e-5
    s1 = (gamma1 * lax.rsqrt(rv1 + eps)).reshape(1, F)
    c1 = ((b1 - rm1) * s1[0] + beta1).reshape(1, F)
    s2 = (gamma2 * lax.rsqrt(rv2 + eps)).reshape(1, F)
    c2 = ((b2 - rm2) * s2[0] + beta2).reshape(1, F)

    degp = _deg_kernel(dst_p, ones128, zeros_s)
    h1_raw = _mm(features, W1)
    dis16, h1p = _prep(degp, h1_raw)
    p1 = _scatter_kernel(h1p, src_p, dst_p, zeros_b)
    h2p = _layer(p1, h1p, dis16, s1, c1, W2)
    p2 = _scatter_kernel(h2p, src_p, dst_p, zeros_b)
    gp0, gp1, gh, gd = _gather_kernel(p2, h2p, dis16, bn)
    return _final(gp0, gp1, gh, gd, s2, c2)


# speedup vs baseline: 1.4747x; 1.4747x over previous
import jax
import jax.numpy as jnp
from jax import lax
from jax.experimental import pallas as pl
from jax.experimental.pallas import tpu as pltpu
from jax.experimental.pallas import tpu_sc as plsc

N = 10000
NPAD = 10112
F = 128
E = 320000
NW = 32
GROUPS = 79
EPW = GROUPS * 128
EPAD = EPW * NW
RPT = NPAD // 16
NB = 1024
BPW = NB // NW

_mesh = plsc.VectorSubcoreMesh(core_axis_name="c", subcore_axis_name="s")
_f32 = jnp.float32


def _deg_body(dst_hbm, ones_hbm, zeros_hbm, out_hbm, idx_v, ones_v, acc_sh):
    c = lax.axis_index("c")
    s = lax.axis_index("s")
    wid = s * 2 + c
    pltpu.sync_copy(zeros_hbm.at[pl.ds(s * RPT, RPT)],
                    acc_sh.at[pl.ds(s * RPT, RPT)])
    pltpu.sync_copy(ones_hbm, ones_v)
    pltpu.sync_copy(dst_hbm.at[wid], idx_v)
    plsc.subcore_barrier()

    def body(g, carry):
        pltpu.sync_copy(ones_v, acc_sh.at[idx_v.at[g]], add=True)
        return carry

    lax.fori_loop(0, GROUPS, body, 0)
    plsc.subcore_barrier()
    pltpu.sync_copy(acc_sh.at[pl.ds(s * RPT, RPT)],
                    out_hbm.at[c].at[pl.ds(s * RPT, RPT)])


_deg_kernel = pl.kernel(
    _deg_body,
    out_type=jax.ShapeDtypeStruct((2, NPAD, F), _f32),
    mesh=_mesh,
    scratch_types=[
        pltpu.VMEM((GROUPS, 128), jnp.int32),
        pltpu.VMEM((128, F), _f32),
        pltpu.VMEM_SHARED((NPAD, F), _f32),
    ],
)


def _scatter_body(h_hbm, src_hbm, dst_hbm, zeros_hbm, out_hbm,
                  src_v, dst_v, rows_v, acc_sh, sem):
    c = lax.axis_index("c")
    s = lax.axis_index("s")
    wid = s * 2 + c
    pltpu.sync_copy(zeros_hbm.at[pl.ds(s * RPT, RPT)],
                    acc_sh.at[pl.ds(s * RPT, RPT)])
    pltpu.sync_copy(src_hbm.at[wid], src_v)
    pltpu.sync_copy(dst_hbm.at[wid], dst_v)
    plsc.subcore_barrier()

    def body(g, carry):
        pltpu.async_copy(h_hbm.at[src_v.at[g]], rows_v, sem).wait()
        pltpu.sync_copy(rows_v, acc_sh.at[dst_v.at[g]], add=True)
        return carry

    lax.fori_loop(0, GROUPS, body, 0)
    plsc.subcore_barrier()
    pltpu.sync_copy(acc_sh.at[pl.ds(s * RPT, RPT)],
                    out_hbm.at[pl.ds(s * RPT, RPT), pl.ds(c * F, F)])


_scatter_kernel = pl.kernel(
    _scatter_body,
    out_type=jax.ShapeDtypeStruct((NPAD, 2 * F), _f32),
    mesh=_mesh,
    scratch_types=[
        pltpu.VMEM((GROUPS, 128), jnp.int32),
        pltpu.VMEM((GROUPS, 128), jnp.int32),
        pltpu.VMEM((128, F), _f32),
        pltpu.VMEM_SHARED((NPAD, F), _f32),
        pltpu.SemaphoreType.DMA,
    ],
)


def _gather_body(p2_hbm, h2_hbm, dis_hbm, bn_hbm, gp_hbm, gh_hbm, gd_hbm,
                 idx_v, bp, bh, bd, sem):
    c = lax.axis_index("c")
    s = lax.axis_index("s")
    wid = s * 2 + c
    pltpu.sync_copy(bn_hbm.at[wid], idx_v)
    pltpu.async_copy(p2_hbm.at[idx_v.at[0]], bp, sem).wait()
    pltpu.async_copy(h2_hbm.at[idx_v.at[0]], bh, sem).wait()
    pltpu.async_copy(dis_hbm.at[idx_v.at[0]], bd, sem).wait()
    base = wid * BPW
    pltpu.sync_copy(bp, gp_hbm.at[pl.ds(base, BPW)])
    pltpu.sync_copy(bh, gh_hbm.at[pl.ds(base, BPW)])
    pltpu.sync_copy(bd, gd_hbm.at[pl.ds(base, BPW)])


_gather_kernel = pl.kernel(
    _gather_body,
    out_type=(
        jax.ShapeDtypeStruct((NB, 2 * F), _f32),
        jax.ShapeDtypeStruct((NB, F), _f32),
        jax.ShapeDtypeStruct((NB, F), _f32),
    ),
    mesh=_mesh,
    scratch_types=[
        pltpu.VMEM((1, BPW), jnp.int32),
        pltpu.VMEM((BPW, 2 * F), _f32),
        pltpu.VMEM((BPW, F), _f32),
        pltpu.VMEM((BPW, F), _f32),
        pltpu.SemaphoreType.DMA,
    ],
)


def _mm_body(x_ref, w_ref, o_ref):
    o_ref[...] = jnp.dot(x_ref[...], w_ref[...],
                         preferred_element_type=jnp.float32)


_mm = pl.pallas_call(
    _mm_body,
    out_shape=jax.ShapeDtypeStruct((N, F), _f32),
)


def _prep_body(degp_ref, h_ref, dis_ref, hp_ref):
    deg = degp_ref[0, :N, 0:1] + degp_ref[1, :N, 0:1] + 1.0
    dis = lax.rsqrt(deg)
    dis_ref[...] = jnp.broadcast_to(dis, (N, F))
    hp_ref[...] = h_ref[...] * dis


_prep = pl.pallas_call(
    _prep_body,
    out_shape=(
        jax.ShapeDtypeStruct((N, F), _f32),
        jax.ShapeDtypeStruct((N, F), _f32),
    ),
)


def _layer_body(p_ref, hp_ref, dis_ref, s1_ref, c1_ref, w_ref, o_ref):
    agg = p_ref[:N, :F] + p_ref[:N, F:] + hp_ref[...]
    d = dis_ref[:, 0:1]
    z = agg * d * s1_ref[...] + c1_ref[...]
    x1 = jnp.maximum(z, 0.0)
    h2 = jnp.dot(x1, w_ref[...], preferred_element_type=jnp.float32)
    o_ref[...] = h2 * d


_layer = pl.pallas_call(
    _layer_body,
    out_shape=jax.ShapeDtypeStruct((N, F), _f32),
)


def _final_body(gp_ref, gh_ref, gd_ref, s2_ref, c2_ref, o_ref):
    agg = gp_ref[:, :F] + gp_ref[:, F:] + gh_ref[...]
    z = agg * gd_ref[:, 0:1] * s2_ref[...] + c2_ref[...]
    x2 = jnp.maximum(z, 0.0)
    m = jnp.max(x2, axis=1, keepdims=True)
    e = jnp.exp(x2 - m)
    lse = jnp.log(jnp.sum(e, axis=1, keepdims=True))
    o_ref[...] = (x2 - m) - lse


_final = pl.pallas_call(
    _final_body,
    out_shape=jax.ShapeDtypeStruct((NB, F), _f32),
)


def kernel(features, edge_index, batch_nodes, device,
           W1, b1, gamma1, beta1, rm1, rv1,
           W2, b2, gamma2, beta2, rm2, rv2):
    del device
    pad = EPAD - E
    src_p = jnp.concatenate(
        [edge_index[0], jnp.zeros((pad,), jnp.int32)]).reshape(NW, GROUPS, 128)
    dst_p = jnp.concatenate(
        [edge_index[1], jnp.full((pad,), N, jnp.int32)]).reshape(NW, GROUPS, 128)
    bn = batch_nodes.reshape(NW, 1, BPW)
    ones128 = jnp.ones((128, F), _f32)
    zeros_f = jnp.zeros((NPAD, F), _f32)
    eps = 1e-5
    s1 = (gamma1 * lax.rsqrt(rv1 + eps)).reshape(1, F)
    c1 = ((b1 - rm1) * s1[0] + beta1).reshape(1, F)
    s2 = (gamma2 * lax.rsqrt(rv2 + eps)).reshape(1, F)
    c2 = ((b2 - rm2) * s2[0] + beta2).reshape(1, F)

    degp = _deg_kernel(dst_p, ones128, zeros_f)
    h1_raw = _mm(features, W1)
    dis16, h1p = _prep(degp, h1_raw)
    p1 = _scatter_kernel(h1p, src_p, dst_p, zeros_f)
    h2p = _layer(p1, h1p, dis16, s1, c1, W2)
    p2 = _scatter_kernel(h2p, src_p, dst_p, zeros_f)
    gp, gh, gd = _gather_kernel(p2, h2p, dis16, bn)
    return _final(gp, gh, gd, s2, c2)
